# baseline (speedup 1.0000x reference)
import jax
import jax.numpy as jnp
from jax import lax
from jax.experimental import pallas as pl
from jax.experimental.pallas import tpu as pltpu
from jax.experimental.pallas import tpu_sc as plsc

_EPS = 1e-8
_D = 256

_SC_W = 32
_SC_CH = 160
_SC_NCH = 8
_SC_RPW = _SC_CH * _SC_NCH
_SC_ROWS = _SC_W * _SC_RPW
_TC_ROWS = 100000 - _SC_ROWS
_TC_NB = 6
_TC_BLK = _TC_ROWS // _TC_NB


def _tc_block(e_ref, x_ref, o_ref):
    x = x_ref[:]
    e = e_ref[:]
    dims = (((1,), (1,)), ((), ()))
    num = jax.lax.dot_general(e, x, dims,
                              preferred_element_type=jnp.float32)
    ones = jnp.ones((1, _D), jnp.float32)
    n2sq = jax.lax.dot_general(ones, x * x, dims,
                               preferred_element_type=jnp.float32)
    n2 = jnp.sqrt(n2sq)
    n1 = jnp.sqrt(jnp.sum(e * e))
    denom = jnp.maximum(n1, _EPS) * jnp.maximum(n2, _EPS)
    o_ref[0] = num / denom


def _tc_part(embedding, x):
    out = pl.pallas_call(
        _tc_block,
        grid=(_TC_NB,),
        in_specs=[
            pl.BlockSpec((1, _D), lambda i: (0, 0)),
            pl.BlockSpec((_TC_BLK, _D), lambda i: (i, 0)),
        ],
        out_specs=pl.BlockSpec((1, 1, _TC_BLK), lambda i: (i, 0, 0)),
        out_shape=jax.ShapeDtypeStruct((_TC_NB, 1, _TC_BLK), jnp.float32),
    )(embedding, x)
    return out.reshape(_TC_ROWS)


def _nrsqrt(v):
    i = plsc.bitcast(v, jnp.int32)
    i = 0x5F3759DF - (i >> 1)
    y = plsc.bitcast(i, jnp.float32)
    for _ in range(3):
        y = y * (1.5 - 0.5 * v * y * y)
    return jnp.minimum(y, 1.0 / _EPS)


def _sc_body(e_hbm, x_hbm, o_hbm, e_v, x_v, o_v, sem):
    c = lax.axis_index("c")
    s = lax.axis_index("s")
    wid = s * 2 + c
    base = _TC_ROWS + wid * _SC_RPW

    pltpu.sync_copy(e_hbm, e_v)

    ev = [e_v[pl.ds(j * 16, 16)] for j in range(_D // 16)]

    eacc = jnp.zeros((16,), jnp.float32)
    for v in ev:
        eacc = eacc + v * v
    s1 = plsc.cumsum(eacc)[jnp.full((16,), 15, jnp.int32)]
    inv1 = _nrsqrt(s1)

    lane = lax.iota(jnp.int32, 16)

    def chunk_src(ch):
        return x_hbm.at[pl.ds(base + ch * _SC_CH, _SC_CH)]

    def buf_dst(ch):
        return x_v.at[pl.ds((ch % 2) * _SC_CH, _SC_CH)]

    pltpu.async_copy(chunk_src(0), buf_dst(0), sem)

    def chunk_body(ch, _):
        pltpu.make_async_copy(chunk_src(ch), buf_dst(ch), sem).wait()

        @pl.when(ch + 1 < _SC_NCH)
        def _start_next():
            pltpu.async_copy(chunk_src(ch + 1), buf_dst(ch + 1), sem)

        boff = (ch % 2) * _SC_CH

        def group_body(g, _):
            rows = boff + g * 16 + lane
            def j_body(j2, accs):
                ad = list(accs[:4])
                as_ = list(accs[4:])
                for u in range(2):
                    dcol = jnp.bitwise_and(lane + (j2 * 2 + u), 15)
                    for t in range(_D // 16):
                        col = plsc.load_gather(x_v, [rows, dcol + (t * 16)])
                        b = t % 4
                        ad[b] = ad[b] + col * ev[t][dcol]
                        as_[b] = as_[b] + col * col
                return tuple(ad) + tuple(as_)

            z = jnp.zeros((16,), jnp.float32)
            accs = lax.fori_loop(0, 8, j_body, (z,) * 8)
            ad = list(accs[:4])
            as_ = list(accs[4:])
            for st in (2, 1):
                for b in range(st):
                    ad[b] = ad[b] + ad[b + st]
                    as_[b] = as_[b] + as_[b + st]
            sim = ad[0] * _nrsqrt(as_[0]) * inv1
            o_v[pl.ds(ch * _SC_CH + g * 16, 16)] = sim
            return 0

        lax.fori_loop(0, _SC_CH // 16, group_body, 0)
        return 0

    lax.fori_loop(0, _SC_NCH, chunk_body, 0)

    pltpu.sync_copy(o_v, o_hbm.at[pl.ds(wid * _SC_RPW, _SC_RPW)])


def _sc_part(embedding, x):
    mesh = plsc.VectorSubcoreMesh(core_axis_name="c", subcore_axis_name="s")
    fn = pl.kernel(
        _sc_body,
        out_type=jax.ShapeDtypeStruct((_SC_ROWS,), jnp.float32),
        mesh=mesh,
        compiler_params=pltpu.CompilerParams(needs_layout_passes=False),
        scratch_types=[
            pltpu.VMEM((_D,), jnp.float32),
            pltpu.VMEM((2 * _SC_CH, _D), jnp.float32),
            pltpu.VMEM((_SC_RPW,), jnp.float32),
            pltpu.SemaphoreType.DMA,
        ],
    )
    return fn(embedding.reshape(_D), x)


def kernel(embedding, x):
    out_sc = _sc_part(embedding, x)
    out_tc = _tc_part(embedding, x)
    return jnp.concatenate([out_tc, out_sc])


# speedup vs baseline: 1.2866x; 1.2866x over previous
import jax
import jax.numpy as jnp
from jax import lax
from jax.experimental import pallas as pl
from jax.experimental.pallas import tpu as pltpu
from jax.experimental.pallas import tpu_sc as plsc

_EPS = 1e-8
_D = 256

_SC_W = 32
_SC_CH = 160
_SC_NCH = 8
_SC_RPW = _SC_CH * _SC_NCH
_SC_ROWS = _SC_W * _SC_RPW
_TC_ROWS = 100000 - _SC_ROWS
_TC_NB = 6
_TC_BLK = _TC_ROWS // _TC_NB


def _tc_block(e_ref, x_ref, o_ref):
    x = x_ref[:]
    e = e_ref[:]
    dims = (((1,), (1,)), ((), ()))
    num = jax.lax.dot_general(e, x, dims,
                              preferred_element_type=jnp.float32)
    ones = jnp.ones((1, _D), jnp.float32)
    n2sq = jax.lax.dot_general(ones, x * x, dims,
                               preferred_element_type=jnp.float32)
    n2 = jnp.sqrt(n2sq)
    n1 = jnp.sqrt(jnp.sum(e * e))
    denom = jnp.maximum(n1, _EPS) * jnp.maximum(n2, _EPS)
    o_ref[0] = num / denom


def _tc_part(embedding, x):
    out = pl.pallas_call(
        _tc_block,
        grid=(_TC_NB,),
        in_specs=[
            pl.BlockSpec((1, _D), lambda i: (0, 0)),
            pl.BlockSpec((_TC_BLK, _D), lambda i: (i, 0)),
        ],
        out_specs=pl.BlockSpec((1, 1, _TC_BLK), lambda i: (i, 0, 0)),
        out_shape=jax.ShapeDtypeStruct((_TC_NB, 1, _TC_BLK), jnp.float32),
    )(embedding, x)
    return out.reshape(_TC_ROWS)


def _nrsqrt(v):
    i = plsc.bitcast(v, jnp.int32)
    i = 0x5F3759DF - (i >> 1)
    y = plsc.bitcast(i, jnp.float32)
    for _ in range(3):
        y = y * (1.5 - 0.5 * v * y * y)
    return jnp.minimum(y, 1.0 / _EPS)


def _sc_body(e_hbm, x_hbm, o_hbm, e_v, x_v, o_v, sem):
    c = lax.axis_index("c")
    s = lax.axis_index("s")
    wid = s * 2 + c
    base = _TC_ROWS + wid * _SC_RPW

    pltpu.sync_copy(e_hbm, e_v)

    ev = [e_v[pl.ds(j * 16, 16)] for j in range(_D // 16)]

    eacc = jnp.zeros((16,), jnp.float32)
    for v in ev:
        eacc = eacc + v * v
    s1 = plsc.cumsum(eacc)[jnp.full((16,), 15, jnp.int32)]
    inv1 = _nrsqrt(s1)

    lane = lax.iota(jnp.int32, 16)

    def chunk_src(ch):
        return x_hbm.at[pl.ds(base + ch * _SC_CH, _SC_CH)]

    def buf_dst(ch):
        return x_v.at[pl.ds((ch % 2) * _SC_CH, _SC_CH)]

    pltpu.async_copy(chunk_src(0), buf_dst(0), sem)

    def chunk_body(ch, _):
        pltpu.make_async_copy(chunk_src(ch), buf_dst(ch), sem).wait()

        @pl.when(ch + 1 < _SC_NCH)
        def _start_next():
            pltpu.async_copy(chunk_src(ch + 1), buf_dst(ch + 1), sem)

        boff = (ch % 2) * _SC_CH

        def group_body(g, _):
            rows = boff + g * 16 + lane
            def j_body(j, accs):
                ad = list(accs[:4])
                as_ = list(accs[4:])
                dcol = jnp.bitwise_and(lane + j, 15)
                for t in range(_D // 16):
                    col = plsc.load_gather(x_v, [rows, dcol + (t * 16)])
                    b = t % 4
                    ad[b] = ad[b] + col * ev[t][dcol]
                    as_[b] = as_[b] + col * col
                return tuple(ad) + tuple(as_)

            z = jnp.zeros((16,), jnp.float32)
            accs = lax.fori_loop(0, 16, j_body, (z,) * 8)
            ad = list(accs[:4])
            as_ = list(accs[4:])
            for st in (2, 1):
                for b in range(st):
                    ad[b] = ad[b] + ad[b + st]
                    as_[b] = as_[b] + as_[b + st]
            sim = ad[0] * _nrsqrt(as_[0]) * inv1
            o_v[pl.ds(ch * _SC_CH + g * 16, 16)] = sim
            return 0

        lax.fori_loop(0, _SC_CH // 16, group_body, 0)
        return 0

    lax.fori_loop(0, _SC_NCH, chunk_body, 0)

    pltpu.sync_copy(o_v, o_hbm.at[pl.ds(wid * _SC_RPW, _SC_RPW)])


def _sc_part(embedding, x):
    mesh = plsc.VectorSubcoreMesh(core_axis_name="c", subcore_axis_name="s")
    fn = pl.kernel(
        _sc_body,
        out_type=jax.ShapeDtypeStruct((_SC_ROWS,), jnp.float32),
        mesh=mesh,
        compiler_params=pltpu.CompilerParams(needs_layout_passes=False),
        scratch_types=[
            pltpu.VMEM((_D,), jnp.float32),
            pltpu.VMEM((2 * _SC_CH, _D), jnp.float32),
            pltpu.VMEM((_SC_RPW,), jnp.float32),
            pltpu.SemaphoreType.DMA,
        ],
    )
    return fn(embedding.reshape(_D), x)


def kernel(embedding, x):
    out_sc = _sc_part(embedding, x)
    out_tc = _tc_part(embedding, x)
    return jnp.concatenate([out_tc, out_sc])
